# baseline (speedup 1.0000x reference)
import functools

import jax
import jax.numpy as jnp
from jax import lax
from jax.experimental import pallas as pl
from jax.experimental.pallas import tpu as pltpu
from jax.experimental.pallas import tpu_sc as plsc

_NUM_FIELDS = 26
_VOCAB = 100000
_EMBED_DIM = 32
_BATCH = 16384
_NPLANE = _NUM_FIELDS * _EMBED_DIM
_BCH = 4096
_NCH = _BATCH // _BCH
_UNROLL = 8
_NIT = _BCH // (16 * _UNROLL)

_mesh = plsc.VectorSubcoreMesh(core_axis_name="c", subcore_axis_name="s")


@functools.partial(
    pl.kernel,
    mesh=_mesh,
    out_type=jax.ShapeDtypeStruct((_NPLANE, _BATCH), jnp.float32),
    scratch_types=[
        pltpu.VMEM((_VOCAB,), jnp.float32),
        pltpu.VMEM((_BCH,), jnp.int32),
        pltpu.VMEM((_BCH,), jnp.int32),
        pltpu.VMEM((_BCH,), jnp.float32),
        pltpu.VMEM((_BCH,), jnp.float32),
        pltpu.VMEM((_BCH,), jnp.float32),
        pltpu.VMEM((_BCH,), jnp.float32),
        pltpu.VMEM_SHARED((4, _BATCH), jnp.int32),
        pltpu.SemaphoreType.DMA,
        pltpu.SemaphoreType.DMA,
        pltpu.SemaphoreType.DMA,
        pltpu.SemaphoreType.DMA,
        pltpu.SemaphoreType.DMA,
        pltpu.SemaphoreType.DMA,
        pltpu.SemaphoreType.DMA,
        pltpu.SemaphoreType.DMA,
    ],
    compiler_params=pltpu.CompilerParams(
        use_tc_tiling_on_sc=True, needs_layout_passes=False
    ),
)
def _emb_lookup(xt_hbm, t2_hbm, out_hbm,
                row_v, idx_a, idx_b, v0, v1, v2, v3, xwin,
                s_row, s_xa, s_xb, s_o0, s_o1, s_o2, s_o3, s_st):
    sid = lax.axis_index("s")
    wid = sid * 2 + lax.axis_index("c")
    idx_bufs = ((idx_a, s_xa), (idx_b, s_xb))
    val_bufs = ((v0, s_o0), (v1, s_o1), (v2, s_o2), (v3, s_o3))

    def gather_chunk(ib, vb):
        def body(i, carry):
            base = i * (16 * _UNROLL)
            for u in range(_UNROLL):
                sl = pl.ds(base + u * 16, 16)
                vb[sl] = plsc.load_gather(row_v, [ib[sl]])
            return carry

        lax.fori_loop(0, _NIT, body, 0)

    h_row = pltpu.async_copy(t2_hbm.at[wid], row_v, s_row)

    @pl.when(sid == 0)
    def _():
        pltpu.sync_copy(xt_hbm.at[0], xwin.at[0])
        pltpu.sync_copy(xt_hbm.at[1], xwin.at[1])

    plsc.subcore_barrier()
    h_x = pltpu.async_copy(xwin.at[0, pl.ds(0, _BCH)], idx_a, s_xa)
    out_h = [None, None, None, None]

    for j in range(_NUM_FIELDS):
        p = j * _EMBED_DIM + wid
        if j % 2 == 0 and j + 2 < _NUM_FIELDS:

            @pl.when(sid == 0)
            def _():
                pltpu.async_copy(xt_hbm.at[j + 2], xwin.at[(j + 2) % 4], s_st)
                if j + 3 < _NUM_FIELDS:
                    pltpu.async_copy(
                        xt_hbm.at[j + 3], xwin.at[(j + 3) % 4], s_st)

        h_row.wait()
        for c in range(_NCH):
            ib, _ = idx_bufs[c % 2]
            vb, s_v = val_bufs[c]
            h_x.wait()
            if c + 1 < _NCH:
                nib, ns = idx_bufs[(c + 1) % 2]
                h_x = pltpu.async_copy(
                    xwin.at[j % 4, pl.ds((c + 1) * _BCH, _BCH)], nib, ns)
            if out_h[c] is not None:
                out_h[c].wait()
            gather_chunk(ib, vb)
            out_h[c] = pltpu.async_copy(
                vb, out_hbm.at[p, pl.ds(c * _BCH, _BCH)], s_v)
        if j + 1 < _NUM_FIELDS:
            h_row = pltpu.async_copy(
                t2_hbm.at[(j + 1) * _EMBED_DIM + wid], row_v, s_row)

            if j % 2 == 1:
                if j + 1 < _NUM_FIELDS:

                    @pl.when(sid == 0)
                    def _():
                        pltpu.make_async_copy(
                            xt_hbm.at[j + 1], xwin.at[(j + 1) % 4],
                            s_st).wait()
                        if j + 2 < _NUM_FIELDS:
                            pltpu.make_async_copy(
                                xt_hbm.at[j + 2], xwin.at[(j + 2) % 4],
                                s_st).wait()

                plsc.subcore_barrier()
            nib, ns = idx_bufs[0]
            h_x = pltpu.async_copy(
                xwin.at[(j + 1) % 4, pl.ds(0, _BCH)], nib, ns)

    for h in out_h:
        h.wait()


def kernel(x, tables):
    xt = x.astype(jnp.int32).T
    t2 = tables.transpose(0, 2, 1).reshape(_NPLANE, _VOCAB)
    out = _emb_lookup(xt, t2)
    return out.reshape(_NUM_FIELDS, _EMBED_DIM, _BATCH).transpose(2, 0, 1)


# speedup vs baseline: 1.0006x; 1.0006x over previous
import functools

import jax
import jax.numpy as jnp
from jax import lax
from jax.experimental import pallas as pl
from jax.experimental.pallas import tpu as pltpu
from jax.experimental.pallas import tpu_sc as plsc

_NUM_FIELDS = 26
_VOCAB = 100000
_EMBED_DIM = 32
_BATCH = 16384
_NPLANE = _NUM_FIELDS * _EMBED_DIM
_BCH = 4096
_NCH = _BATCH // _BCH
_UNROLL = 8
_NIT = _BCH // (16 * _UNROLL)

_mesh = plsc.VectorSubcoreMesh(core_axis_name="c", subcore_axis_name="s")


@functools.partial(
    pl.kernel,
    mesh=_mesh,
    out_type=jax.ShapeDtypeStruct((_NPLANE, _BATCH), jnp.float32),
    scratch_types=[
        pltpu.VMEM((_VOCAB,), jnp.float32),
        pltpu.VMEM((_BCH,), jnp.int32),
        pltpu.VMEM((_BCH,), jnp.int32),
        pltpu.VMEM((_BCH,), jnp.float32),
        pltpu.VMEM((_BCH,), jnp.float32),
        pltpu.VMEM((_BCH,), jnp.float32),
        pltpu.VMEM((_BCH,), jnp.float32),
        pltpu.VMEM_SHARED((2, _BATCH), jnp.int32),
        pltpu.SemaphoreType.DMA,
        pltpu.SemaphoreType.DMA,
        pltpu.SemaphoreType.DMA,
        pltpu.SemaphoreType.DMA,
        pltpu.SemaphoreType.DMA,
        pltpu.SemaphoreType.DMA,
        pltpu.SemaphoreType.DMA,
        pltpu.SemaphoreType.DMA,
    ],
    compiler_params=pltpu.CompilerParams(
        use_tc_tiling_on_sc=True, needs_layout_passes=False
    ),
)
def _emb_lookup(xt_hbm, t2_hbm, out_hbm,
                row_v, idx_a, idx_b, v0, v1, v2, v3, xwin,
                s_row, s_xa, s_xb, s_o0, s_o1, s_o2, s_o3, s_st):
    sid = lax.axis_index("s")
    wid = sid * 2 + lax.axis_index("c")
    idx_bufs = ((idx_a, s_xa), (idx_b, s_xb))
    val_bufs = ((v0, s_o0), (v1, s_o1), (v2, s_o2), (v3, s_o3))

    def gather_chunk(ib, vb):
        def body(i, carry):
            base = i * (16 * _UNROLL)
            for u in range(_UNROLL):
                sl = pl.ds(base + u * 16, 16)
                vb[sl] = plsc.load_gather(row_v, [ib[sl]])
            return carry

        lax.fori_loop(0, _NIT, body, 0)

    h_row = pltpu.async_copy(t2_hbm.at[wid], row_v, s_row)

    @pl.when(sid == 0)
    def _():
        pltpu.sync_copy(xt_hbm.at[0], xwin.at[0])

    plsc.subcore_barrier()
    h_x = pltpu.async_copy(xwin.at[0, pl.ds(0, _BCH)], idx_a, s_xa)
    out_h = [None, None, None, None]

    for j in range(_NUM_FIELDS):
        p = j * _EMBED_DIM + wid
        if j + 1 < _NUM_FIELDS:

            @pl.when(sid == 0)
            def _():
                pltpu.async_copy(xt_hbm.at[j + 1], xwin.at[(j + 1) % 2], s_st)

        h_row.wait()
        for c in range(_NCH):
            ib, _ = idx_bufs[c % 2]
            vb, s_v = val_bufs[c]
            h_x.wait()
            if c + 1 < _NCH:
                nib, ns = idx_bufs[(c + 1) % 2]
                h_x = pltpu.async_copy(
                    xwin.at[j % 2, pl.ds((c + 1) * _BCH, _BCH)], nib, ns)
            if out_h[c] is not None:
                out_h[c].wait()
            gather_chunk(ib, vb)
            out_h[c] = pltpu.async_copy(
                vb, out_hbm.at[p, pl.ds(c * _BCH, _BCH)], s_v)
        if j + 1 < _NUM_FIELDS:
            h_row = pltpu.async_copy(
                t2_hbm.at[(j + 1) * _EMBED_DIM + wid], row_v, s_row)

            @pl.when(sid == 0)
            def _():
                pltpu.make_async_copy(
                    xt_hbm.at[j + 1], xwin.at[(j + 1) % 2], s_st).wait()

            plsc.subcore_barrier()
            nib, ns = idx_bufs[0]
            h_x = pltpu.async_copy(
                xwin.at[(j + 1) % 2, pl.ds(0, _BCH)], nib, ns)

    for h in out_h:
        h.wait()


def kernel(x, tables):
    xt = x.astype(jnp.int32).T
    t2 = tables.transpose(0, 2, 1).reshape(_NPLANE, _VOCAB)
    out = _emb_lookup(xt, t2)
    return out.reshape(_NUM_FIELDS, _EMBED_DIM, _BATCH).transpose(2, 0, 1)
